# baseline (speedup 1.0000x reference)
import functools

import jax
import jax.numpy as jnp
from jax import lax
from jax.experimental import pallas as pl
from jax.experimental.pallas import tpu as pltpu
from jax.experimental.pallas import tpu_sc as plsc

N = 10000
D = 128
E = 320000

NC = 2
NS = 16
NW = NC * NS
C = 128
CH = 80
E_PAD = NW * CH * C
ROWS_A = 624


def _mm_body(feat_ref, w_ref, out_ref):
    r = jnp.dot(feat_ref[...], w_ref[...], preferred_element_type=jnp.float32)
    out_ref[...] = r.astype(jnp.bfloat16)


def _support_matmul(feat, W):
    BLK = 1000
    return pl.pallas_call(
        _mm_body,
        grid=(N // BLK,),
        in_specs=[
            pl.BlockSpec((BLK, D), lambda i: (i, 0)),
            pl.BlockSpec((D, D), lambda i: (0, 0)),
        ],
        out_specs=pl.BlockSpec((BLK, D), lambda i: (i, 0)),
        out_shape=jax.ShapeDtypeStruct((N, D), jnp.bfloat16),
    )(feat, W)


def _sc_body(sup_hbm, srcb_hbm, dstb_hbm, ewb_hbm, zeros_hbm, out_hbm,
             src_v, dst_v, rows0, rows1, rowsq0, rowsq1, ew_v, acc,
             semr0, semr1, semw0, semw1):
    cid = lax.axis_index("c")
    sid = lax.axis_index("s")
    wid = cid * NS + sid

    pltpu.sync_copy(srcb_hbm.at[wid], src_v)
    pltpu.sync_copy(dstb_hbm.at[wid], dst_v)
    pltpu.sync_copy(ewb_hbm.at[wid], ew_v)

    pltpu.sync_copy(zeros_hbm, acc.at[pl.ds(sid * ROWS_A, ROWS_A)])

    @pl.when(sid == NS - 1)
    def _():
        pltpu.sync_copy(zeros_hbm.at[pl.ds(0, 16)], acc.at[pl.ds(NS * ROWS_A, 16)])

    plsc.subcore_barrier()

    rows = (rows0, rows1)
    rowsq = (rowsq0, rowsq1)
    semr = (semr0, semr1)
    semw = (semw0, semw1)

    def _issue(kk, b):
        pltpu.async_copy(sup_hbm.at[src_v.at[kk]], rows[b], semr[b])

    _issue(0, 0)
    _issue(1, 1)

    RND = jnp.float32(12582912.0)

    def _scale_group(g, carry, b, kk):
        w16 = ew_v[kk, pl.ds(g * 16, 16)]
        for u in range(16):
            wb = lax.gather(
                w16, jnp.full((16, 1), u, jnp.int32),
                lax.GatherDimensionNumbers(
                    offset_dims=(), collapsed_slice_dims=(0,),
                    start_index_map=(0,)),
                (1,), mode=lax.GatherScatterMode.PROMISE_IN_BOUNDS)
            e = g * 16 + u
            for fb in range(D // 32):
                x = plsc.bitcast(rows[b][e, pl.ds(fb * 32, 32)], jnp.int32)
                lo = plsc.bitcast(jnp.left_shift(x, 16), jnp.float32)
                hi = plsc.bitcast(jnp.bitwise_and(x, jnp.int32(-65536)),
                                  jnp.float32)
                ai = ((lo * wb + RND) - RND).astype(jnp.int32)
                bi = ((hi * wb + RND) - RND).astype(jnp.int32)
                q = plsc.pack(ai, bi, format=plsc.PackFormat.INTERLEAVED,
                              preferred_element_type=jnp.int16)
                rowsq[b][e, pl.ds(fb * 32, 32)] = q
        return carry

    def _outer(i, carry):
        k = i * 2
        for b in (0, 1):
            kk = k + b
            pltpu.make_async_copy(sup_hbm.at[src_v.at[kk]], rows[b],
                                  semr[b]).wait()

            @pl.when(kk >= 2)
            def _():
                pltpu.make_async_copy(
                    rowsq[b], acc.at[dst_v.at[kk - 2]], semw[b]).wait()

            lax.fori_loop(0, C // 16,
                          functools.partial(_scale_group, b=b, kk=kk), 0)
            pltpu.async_copy(rowsq[b], acc.at[dst_v.at[kk]], semw[b], add=True)

            @pl.when(kk + 2 < CH)
            def _():
                _issue(kk + 2, b)
        return carry

    lax.fori_loop(0, CH // 2, _outer, 0)
    pltpu.make_async_copy(rowsq[0], acc.at[dst_v.at[CH - 2]], semw[0]).wait()
    pltpu.make_async_copy(rowsq[1], acc.at[dst_v.at[CH - 1]], semw[1]).wait()
    plsc.subcore_barrier()

    sl = pl.ds(sid * ROWS_A, ROWS_A)
    pltpu.sync_copy(acc.at[sl], out_hbm.at[cid, sl])

    @pl.when(sid == NS - 1)
    def _():
        tl = pl.ds(NS * ROWS_A, 16)
        pltpu.sync_copy(acc.at[tl], out_hbm.at[cid, tl])


def _sc_aggregate(support, srcb, dstb, ewb, zeros):
    mesh = plsc.VectorSubcoreMesh(core_axis_name="c", subcore_axis_name="s")
    f = pl.kernel(
        _sc_body,
        out_type=jax.ShapeDtypeStruct((NC, N, D), jnp.int16),
        mesh=mesh,
        compiler_params=pltpu.CompilerParams(use_tc_tiling_on_sc=False,
                                             needs_layout_passes=False),
        scratch_types=[
            pltpu.VMEM((CH, C), jnp.int32),
            pltpu.VMEM((CH, C), jnp.int32),
            pltpu.VMEM((C, D), jnp.bfloat16),
            pltpu.VMEM((C, D), jnp.bfloat16),
            pltpu.VMEM((C, D), jnp.int16),
            pltpu.VMEM((C, D), jnp.int16),
            pltpu.VMEM((CH, C), jnp.float32),
            pltpu.VMEM_SHARED((N, D), jnp.int16),
            pltpu.SemaphoreType.DMA,
            pltpu.SemaphoreType.DMA,
            pltpu.SemaphoreType.DMA,
            pltpu.SemaphoreType.DMA,
        ],
    )
    return f(support, srcb, dstb, ewb, zeros)


def _fin_body(p_ref, perm_ref, b_ref, out_ref):
    s = p_ref[0].astype(jnp.float32) + p_ref[1].astype(jnp.float32)
    x = jnp.dot(s, perm_ref[...], preferred_element_type=jnp.float32)
    out_ref[...] = jnp.maximum(x + b_ref[...], 0.0)


def _finalize(partials, Pmat, b):
    BLK = 1000
    return pl.pallas_call(
        _fin_body,
        grid=(N // BLK,),
        in_specs=[
            pl.BlockSpec((NC, BLK, D), lambda i: (0, i, 0)),
            pl.BlockSpec((D, D), lambda i: (0, 0)),
            pl.BlockSpec((D,), lambda i: (0,)),
        ],
        out_specs=pl.BlockSpec((BLK, D), lambda i: (i, 0)),
        out_shape=jax.ShapeDtypeStruct((N, D), jnp.float32),
    )(partials, Pmat, b)


def kernel(feat, edge_index, edge_weight, W, b):
    perm = jnp.array([g * 32 + (j % 2) * 16 + j // 2
                      for g in range(D // 32) for j in range(32)],
                     dtype=jnp.int32)
    support = _support_matmul(feat, W[:, perm])

    src = edge_index[0].astype(jnp.int32)
    dst = edge_index[1].astype(jnp.int32)
    ew = edge_weight.astype(jnp.float32)

    pad = E_PAD - E
    srcb = jnp.pad(src, (0, pad)).reshape(NW, CH, C)
    dstb = jnp.pad(dst, (0, pad)).reshape(NW, CH, C)
    ewb = (jnp.pad(ew, (0, pad)) * 256.0).reshape(NW, CH, C)
    zeros = jnp.zeros((ROWS_A, D), jnp.int16)

    stored = [g * 32 + (2 * i if i < 16 else 2 * (i - 16) + 1)
              for g in range(D // 32) for i in range(32)]
    Pmat = jnp.zeros((D, D), jnp.float32).at[
        jnp.array(stored, jnp.int32), jnp.arange(D)].set(1.0 / 256.0)

    partials = _sc_aggregate(support, srcb, dstb, ewb, zeros)
    return _finalize(partials, Pmat, b)


# speedup vs baseline: 1.2034x; 1.2034x over previous
import functools

import jax
import jax.numpy as jnp
from jax import lax
from jax.experimental import pallas as pl
from jax.experimental.pallas import tpu as pltpu
from jax.experimental.pallas import tpu_sc as plsc

N = 10000
D = 128
E = 320000

NC = 2
NS = 16
DH = D // NC
C = 128
CH = 160
E_PAD = NS * CH * C
ROWS_A = 624


def _mm_body(feat_ref, w_ref, out_ref):
    r = jnp.dot(feat_ref[...], w_ref[...], preferred_element_type=jnp.float32)
    out_ref[0] = r[:, :DH].astype(jnp.bfloat16)
    out_ref[1] = r[:, DH:].astype(jnp.bfloat16)


def _support_matmul(feat, W):
    BLK = 1000
    return pl.pallas_call(
        _mm_body,
        grid=(N // BLK,),
        in_specs=[
            pl.BlockSpec((BLK, D), lambda i: (i, 0)),
            pl.BlockSpec((D, D), lambda i: (0, 0)),
        ],
        out_specs=pl.BlockSpec((NC, BLK, DH), lambda i: (0, i, 0)),
        out_shape=jax.ShapeDtypeStruct((NC, N, DH), jnp.bfloat16),
    )(feat, W)


def _sc_body(sup_hbm, srcb_hbm, dstb_hbm, ewb_hbm, zeros_hbm, out_hbm,
             src_v, dst_v, rows0, rows1, rowsq0, rowsq1, ew_v, acc,
             semr0, semr1, semw0, semw1):
    cid = lax.axis_index("c")
    sid = lax.axis_index("s")

    pltpu.sync_copy(srcb_hbm.at[sid], src_v)
    pltpu.sync_copy(dstb_hbm.at[sid], dst_v)
    pltpu.sync_copy(ewb_hbm.at[sid], ew_v)

    pltpu.sync_copy(zeros_hbm, acc.at[pl.ds(sid * ROWS_A, ROWS_A)])

    @pl.when(sid == NS - 1)
    def _():
        pltpu.sync_copy(zeros_hbm.at[pl.ds(0, 16)], acc.at[pl.ds(NS * ROWS_A, 16)])

    plsc.subcore_barrier()

    sup = sup_hbm.at[cid]
    rows = (rows0, rows1)
    rowsq = (rowsq0, rowsq1)
    semr = (semr0, semr1)
    semw = (semw0, semw1)

    def _issue(kk, b):
        pltpu.async_copy(sup.at[src_v.at[kk]], rows[b], semr[b])

    _issue(0, 0)
    _issue(1, 1)

    def _scale_group(g, carry, b, kk):
        w16 = ew_v[kk, pl.ds(g * 16, 16)]
        for u in range(16):
            wb = lax.gather(
                w16, jnp.full((16, 1), u, jnp.int32),
                lax.GatherDimensionNumbers(
                    offset_dims=(), collapsed_slice_dims=(0,),
                    start_index_map=(0,)),
                (1,), mode=lax.GatherScatterMode.PROMISE_IN_BOUNDS)
            wb32 = plsc.pack(wb, wb, format=plsc.PackFormat.INTERLEAVED)
            e = g * 16 + u
            for fb in range(DH // 32):
                prod = rows[b][e, pl.ds(fb * 32, 32)] * wb32
                rowsq[b][e, pl.ds(fb * 32, 32)] = prod.astype(jnp.int16)
        return carry

    def _outer(i, carry):
        k = i * 2
        for b in (0, 1):
            kk = k + b
            pltpu.make_async_copy(sup.at[src_v.at[kk]], rows[b], semr[b]).wait()

            @pl.when(kk >= 2)
            def _():
                pltpu.make_async_copy(
                    rowsq[b], acc.at[dst_v.at[kk - 2]], semw[b]).wait()

            lax.fori_loop(0, C // 16,
                          functools.partial(_scale_group, b=b, kk=kk), 0)
            pltpu.async_copy(rowsq[b], acc.at[dst_v.at[kk]], semw[b], add=True)

            @pl.when(kk + 2 < CH)
            def _():
                _issue(kk + 2, b)
        return carry

    lax.fori_loop(0, CH // 2, _outer, 0)
    pltpu.make_async_copy(rowsq[0], acc.at[dst_v.at[CH - 2]], semw[0]).wait()
    pltpu.make_async_copy(rowsq[1], acc.at[dst_v.at[CH - 1]], semw[1]).wait()
    plsc.subcore_barrier()

    sl = pl.ds(sid * ROWS_A, ROWS_A)
    pltpu.sync_copy(acc.at[sl], out_hbm.at[cid, sl])

    @pl.when(sid == NS - 1)
    def _():
        tl = pl.ds(NS * ROWS_A, 16)
        pltpu.sync_copy(acc.at[tl], out_hbm.at[cid, tl])


def _sc_aggregate(support, srcb, dstb, ewb, zeros):
    mesh = plsc.VectorSubcoreMesh(core_axis_name="c", subcore_axis_name="s")
    f = pl.kernel(
        _sc_body,
        out_type=jax.ShapeDtypeStruct((NC, N, DH), jnp.int16),
        mesh=mesh,
        compiler_params=pltpu.CompilerParams(use_tc_tiling_on_sc=False,
                                             needs_layout_passes=False),
        scratch_types=[
            pltpu.VMEM((CH, C), jnp.int32),
            pltpu.VMEM((CH, C), jnp.int32),
            pltpu.VMEM((C, DH), jnp.bfloat16),
            pltpu.VMEM((C, DH), jnp.bfloat16),
            pltpu.VMEM((C, DH), jnp.int16),
            pltpu.VMEM((C, DH), jnp.int16),
            pltpu.VMEM((CH, C), jnp.float32),
            pltpu.VMEM_SHARED((N, DH), jnp.int16),
            pltpu.SemaphoreType.DMA,
            pltpu.SemaphoreType.DMA,
            pltpu.SemaphoreType.DMA,
            pltpu.SemaphoreType.DMA,
        ],
    )
    return f(support, srcb, dstb, ewb, zeros)


def _fin_body(p_ref, b_ref, out_ref):
    full = jnp.concatenate([p_ref[0], p_ref[1]], axis=1).astype(jnp.float32)
    out_ref[...] = jnp.maximum(full * (1.0 / 256.0) + b_ref[...], 0.0)


def _finalize(partials, b):
    BLK = 1000
    return pl.pallas_call(
        _fin_body,
        grid=(N // BLK,),
        in_specs=[
            pl.BlockSpec((NC, BLK, DH), lambda i: (0, i, 0)),
            pl.BlockSpec((D,), lambda i: (0,)),
        ],
        out_specs=pl.BlockSpec((BLK, D), lambda i: (i, 0)),
        out_shape=jax.ShapeDtypeStruct((N, D), jnp.float32),
    )(partials, b)


def kernel(feat, edge_index, edge_weight, W, b):
    support = _support_matmul(feat, W)

    src = edge_index[0].astype(jnp.int32)
    dst = edge_index[1].astype(jnp.int32)
    ew = edge_weight.astype(jnp.float32)

    pad = E_PAD - E
    srcb = jnp.pad(src, (0, pad)).reshape(NS, CH, C)
    dstb = jnp.pad(dst, (0, pad)).reshape(NS, CH, C)
    ewb = (jnp.pad(ew, (0, pad)) * 256.0).reshape(NS, CH, C)
    zeros = jnp.zeros((ROWS_A, DH), jnp.int16)

    partials = _sc_aggregate(support, srcb, dstb, ewb, zeros)
    return _finalize(partials, b)
